# Initial kernel scaffold; baseline (speedup 1.0000x reference)
import jax
import jax.numpy as jnp
from jax.experimental import pallas as pl


def kernel(input_spikes, input_weights):
    raise NotImplementedError("write your pallas kernel here")


# speedup vs baseline: 3.6676x; 3.6676x over previous
import jax
import jax.numpy as jnp
from jax.experimental import pallas as pl

_N = 512
_O = 256
_EPS = 1e-5


def _eq_kernel(s_ref, w_ref, out_ref):
    s_row = s_ref[0]
    w = w_ref[...]
    s_col = jnp.transpose(s_row)

    e1_row = jnp.exp(s_row)
    e2_row = e1_row * e1_row

    jj = jax.lax.broadcasted_iota(jnp.int32, (_N, _N), 1)
    kk = jax.lax.broadcasted_iota(jnp.int32, (_N, _N), 0)
    m = (s_row < s_col) | ((s_row == s_col) & (jj <= kk))
    mf = jnp.where(m, 1.0, 0.0)

    a1 = jnp.dot(mf * e1_row, w, preferred_element_type=jnp.float32,
                 precision=jax.lax.Precision.HIGHEST)
    a2 = jnp.dot(mf * e2_row, w, preferred_element_type=jnp.float32,
                 precision=jax.lax.Precision.HIGHEST)

    next_e1 = jnp.min(
        jnp.where(m, jnp.inf, jnp.broadcast_to(e1_row, (_N, _N))),
        axis=1, keepdims=True)

    disc = a1 * a1 - 4.0 * a2
    valid = disc > 0.0
    sqrt_d = jnp.sqrt(jnp.where(valid, disc, 1.0))
    denom = a1 + sqrt_d
    nz = denom != 0.0
    ratio = (a2 + a2) / jnp.where(nz, denom, 1.0)
    pos = valid & nz & (ratio > 0.0)

    e1_col = jnp.exp(s_col)
    lo = e1_col * (1.0 - _EPS)
    hi = next_e1 * (1.0 + _EPS)
    ok = pos & (ratio >= lo) & (ratio <= hi)
    cand = jnp.where(ok, ratio, jnp.inf)
    out_ref[0] = jnp.log(jnp.min(cand, axis=0, keepdims=True))


def kernel(input_spikes, input_weights):
    batch = input_spikes.shape[0]
    s3 = input_spikes.reshape(batch, 1, _N)
    out = pl.pallas_call(
        _eq_kernel,
        grid=(batch,),
        in_specs=[
            pl.BlockSpec((1, 1, _N), lambda b: (b, 0, 0)),
            pl.BlockSpec((_N, _O), lambda b: (0, 0)),
        ],
        out_specs=pl.BlockSpec((1, 1, _O), lambda b: (b, 0, 0)),
        out_shape=jax.ShapeDtypeStruct((batch, 1, _O), jnp.float32),
    )(s3, input_weights)
    return out.reshape(batch, _O)


# speedup vs baseline: 6.6203x; 1.8051x over previous
import jax
import jax.numpy as jnp
from jax.experimental import pallas as pl

_N = 512
_O = 256
_EPS = 3e-5


def _eq_kernel(s_ref, w_ref, out_ref):
    s_row = s_ref[0]
    w = w_ref[...]
    s_col = jnp.transpose(s_row)

    e1_row = jnp.exp(s_row)
    e1_col = jnp.exp(s_col)

    jj = jax.lax.broadcasted_iota(jnp.int32, (_N, _N), 1)
    kk = jax.lax.broadcasted_iota(jnp.int32, (_N, _N), 0)
    m = (s_row < s_col) | ((s_row == s_col) & (jj <= kk))
    mb = jnp.where(m, 1.0, 0.0).astype(jnp.bfloat16)

    u1 = e1_col * w
    u2 = e1_col * u1
    u1h = u1.astype(jnp.bfloat16)
    u1l = (u1 - u1h.astype(jnp.float32)).astype(jnp.bfloat16)
    u2h = u2.astype(jnp.bfloat16)
    u2l = (u2 - u2h.astype(jnp.float32)).astype(jnp.bfloat16)

    dot = lambda a, b: jnp.dot(a, b, preferred_element_type=jnp.float32)
    a1 = dot(mb, u1h) + dot(mb, u1l)
    a2 = dot(mb, u2h) + dot(mb, u2l)

    next_e1 = jnp.min(
        jnp.where(m, jnp.inf, jnp.broadcast_to(e1_row, (_N, _N))),
        axis=1, keepdims=True)

    disc = a1 * a1 - 4.0 * a2
    valid = disc > 0.0
    sqrt_d = jnp.sqrt(jnp.where(valid, disc, 1.0))
    ratio = 0.5 * (a1 - sqrt_d)
    pos = valid & (ratio > 0.0)

    lo = e1_col * (1.0 - _EPS)
    hi = next_e1 * (1.0 + _EPS)
    ok = pos & (ratio >= lo) & (ratio <= hi)
    cand = jnp.where(ok, ratio, jnp.inf)
    out_ref[0] = jnp.log(jnp.min(cand, axis=0, keepdims=True))


def kernel(input_spikes, input_weights):
    batch = input_spikes.shape[0]
    s3 = input_spikes.reshape(batch, 1, _N)
    out = pl.pallas_call(
        _eq_kernel,
        grid=(batch,),
        in_specs=[
            pl.BlockSpec((1, 1, _N), lambda b: (b, 0, 0)),
            pl.BlockSpec((_N, _O), lambda b: (0, 0)),
        ],
        out_specs=pl.BlockSpec((1, 1, _O), lambda b: (b, 0, 0)),
        out_shape=jax.ShapeDtypeStruct((batch, 1, _O), jnp.float32),
    )(s3, input_weights)
    return out.reshape(batch, _O)


# speedup vs baseline: 7.4314x; 1.1225x over previous
import jax
import jax.numpy as jnp
from jax.experimental import pallas as pl

_N = 512
_O = 256
_BB = 2
_EPS = 3e-5


def _eq_kernel(s_ref, w_ref, tri_ref, out_ref):
    w = w_ref[...]
    tri = tri_ref[...]

    dot = lambda a, b: jnp.dot(a, b, preferred_element_type=jnp.float32)

    for i in range(_BB):
        s_row = s_ref[i]
        s_col = jnp.transpose(s_row)
        e1_row = jnp.exp(s_row)
        e1_col = jnp.transpose(e1_row)

        m = (s_row < s_col) | ((s_row == s_col) & tri)
        mb = jnp.where(m, 1.0, 0.0).astype(jnp.bfloat16)

        u1 = e1_col * w
        u2 = e1_col * u1
        u1h = u1.astype(jnp.bfloat16)
        u1l = (u1 - u1h.astype(jnp.float32)).astype(jnp.bfloat16)
        u2h = u2.astype(jnp.bfloat16)
        u2l = (u2 - u2h.astype(jnp.float32)).astype(jnp.bfloat16)

        a1 = dot(mb, u1h) + dot(mb, u1l)
        a2 = dot(mb, u2h) + dot(mb, u2l)

        next_e1 = jnp.min(
            jnp.where(m, jnp.inf, jnp.broadcast_to(e1_row, (_N, _N))),
            axis=1, keepdims=True)

        disc = a1 * a1 - 4.0 * a2
        valid = disc > 0.0
        sqrt_d = jnp.sqrt(jnp.where(valid, disc, 1.0))
        ratio = 0.5 * (a1 - sqrt_d)

        lo = e1_col * (1.0 - _EPS)
        hi = next_e1 * (1.0 + _EPS)
        ok = valid & (ratio >= lo) & (ratio <= hi)
        cand = jnp.where(ok, ratio, jnp.inf)
        out_ref[i] = jnp.log(jnp.min(cand, axis=0, keepdims=True))


def kernel(input_spikes, input_weights):
    batch = input_spikes.shape[0]
    s3 = input_spikes.reshape(batch, 1, _N)
    idx = jnp.arange(_N)
    tri = idx[None, :] <= idx[:, None]
    out = pl.pallas_call(
        _eq_kernel,
        grid=(batch // _BB,),
        in_specs=[
            pl.BlockSpec((_BB, 1, _N), lambda b: (b, 0, 0)),
            pl.BlockSpec((_N, _O), lambda b: (0, 0)),
            pl.BlockSpec((_N, _N), lambda b: (0, 0)),
        ],
        out_specs=pl.BlockSpec((_BB, 1, _O), lambda b: (b, 0, 0)),
        out_shape=jax.ShapeDtypeStruct((batch, 1, _O), jnp.float32),
    )(s3, input_weights, tri)
    return out.reshape(batch, _O)


# speedup vs baseline: 8.2833x; 1.1146x over previous
import jax
import jax.numpy as jnp
from jax.experimental import pallas as pl

_N = 512
_O = 256
_BB = 4
_EPS = 3e-5


def _eq_kernel(s_ref, w_ref, tri_ref, out_ref):
    w = w_ref[...]
    tri = tri_ref[...]

    dot = lambda a, b: jnp.dot(a, b, preferred_element_type=jnp.float32)

    for i in range(_BB):
        s_row = s_ref[i]
        s_col = jnp.transpose(s_row)
        e1_row = jnp.exp(s_row)
        e1_col = jnp.transpose(e1_row)

        m = (s_row < s_col) | ((s_row == s_col) & tri)
        mb = jnp.where(m, 1.0, 0.0).astype(jnp.bfloat16)

        u1 = e1_col * w
        u2 = e1_col * u1
        u1h = u1.astype(jnp.bfloat16)
        u1l = (u1 - u1h.astype(jnp.float32)).astype(jnp.bfloat16)
        u2h = u2.astype(jnp.bfloat16)
        u2l = (u2 - u2h.astype(jnp.float32)).astype(jnp.bfloat16)

        rh = jnp.concatenate([u1h, u2h], axis=1)
        rl = jnp.concatenate([u1l, u2l], axis=1)
        a12 = dot(mb, rh) + dot(mb, rl)
        a1 = a12[:, :_O]
        a2 = a12[:, _O:]

        next_e1 = jnp.min(
            jnp.where(m, jnp.inf, jnp.broadcast_to(e1_row, (_N, _N))),
            axis=1, keepdims=True)

        disc = a1 * a1 - 4.0 * a2
        valid = disc > 0.0
        sqrt_d = jnp.sqrt(jnp.where(valid, disc, 1.0))
        ratio = 0.5 * (a1 - sqrt_d)

        lo = e1_col * (1.0 - _EPS)
        hi = next_e1 * (1.0 + _EPS)
        ok = valid & (ratio >= lo) & (ratio <= hi)
        cand = jnp.where(ok, ratio, jnp.inf)
        out_ref[i] = jnp.log(jnp.min(cand, axis=0, keepdims=True))


def kernel(input_spikes, input_weights):
    batch = input_spikes.shape[0]
    s3 = input_spikes.reshape(batch, 1, _N)
    idx = jnp.arange(_N)
    tri = idx[None, :] <= idx[:, None]
    out = pl.pallas_call(
        _eq_kernel,
        grid=(batch // _BB,),
        in_specs=[
            pl.BlockSpec((_BB, 1, _N), lambda b: (b, 0, 0)),
            pl.BlockSpec((_N, _O), lambda b: (0, 0)),
            pl.BlockSpec((_N, _N), lambda b: (0, 0)),
        ],
        out_specs=pl.BlockSpec((_BB, 1, _O), lambda b: (b, 0, 0)),
        out_shape=jax.ShapeDtypeStruct((batch, 1, _O), jnp.float32),
    )(s3, input_weights, tri)
    return out.reshape(batch, _O)


# speedup vs baseline: 8.4416x; 1.0191x over previous
import jax
import jax.numpy as jnp
from jax.experimental import pallas as pl

_N = 512
_O = 256
_BB = 8
_EPS = 3e-5


def _eq_kernel(s_ref, w_ref, tri_ref, out_ref):
    w = w_ref[...]
    tri = tri_ref[...]

    dot = lambda a, b: jnp.dot(a, b, preferred_element_type=jnp.float32)

    for i in range(_BB):
        s_row = s_ref[i]
        s_col = jnp.transpose(s_row)
        e1_row = jnp.exp(s_row)
        e1_col = jnp.transpose(e1_row)

        m = (s_row < s_col) | ((s_row == s_col) & tri)
        mb = jnp.where(m, 1.0, 0.0).astype(jnp.bfloat16)

        u1 = e1_col * w
        u2 = e1_col * u1
        u1h = u1.astype(jnp.bfloat16)
        u1l = (u1 - u1h.astype(jnp.float32)).astype(jnp.bfloat16)
        u2h = u2.astype(jnp.bfloat16)
        u2l = (u2 - u2h.astype(jnp.float32)).astype(jnp.bfloat16)

        rh = jnp.concatenate([u1h, u2h], axis=1)
        rl = jnp.concatenate([u1l, u2l], axis=1)
        a12 = dot(mb, rh) + dot(mb, rl)
        a1 = a12[:, :_O]
        a2 = a12[:, _O:]

        next_e1 = jnp.min(
            jnp.where(m, jnp.inf, jnp.broadcast_to(e1_row, (_N, _N))),
            axis=1, keepdims=True)

        disc = a1 * a1 - 4.0 * a2
        valid = disc > 0.0
        sqrt_d = jnp.sqrt(jnp.where(valid, disc, 1.0))
        ratio = 0.5 * (a1 - sqrt_d)

        lo = e1_col * (1.0 - _EPS)
        hi = next_e1 * (1.0 + _EPS)
        ok = valid & (ratio >= lo) & (ratio <= hi)
        cand = jnp.where(ok, ratio, jnp.inf)
        out_ref[i] = jnp.log(jnp.min(cand, axis=0, keepdims=True))


def kernel(input_spikes, input_weights):
    batch = input_spikes.shape[0]
    s3 = input_spikes.reshape(batch, 1, _N)
    idx = jnp.arange(_N)
    tri = idx[None, :] <= idx[:, None]
    out = pl.pallas_call(
        _eq_kernel,
        grid=(batch // _BB,),
        in_specs=[
            pl.BlockSpec((_BB, 1, _N), lambda b: (b, 0, 0)),
            pl.BlockSpec((_N, _O), lambda b: (0, 0)),
            pl.BlockSpec((_N, _N), lambda b: (0, 0)),
        ],
        out_specs=pl.BlockSpec((_BB, 1, _O), lambda b: (b, 0, 0)),
        out_shape=jax.ShapeDtypeStruct((batch, 1, _O), jnp.float32),
    )(s3, input_weights, tri)
    return out.reshape(batch, _O)
